# baseline (speedup 1.0000x reference)
import functools

import jax
import jax.numpy as jnp
from jax import lax
from jax.experimental import pallas as pl
from jax.experimental.pallas import tpu as pltpu
from jax.experimental.pallas import tpu_sc as plsc

VOCAB = 1000000
DIM = 64
B = 16384
K = 20
KP = 24
W2 = 2 * DIM

NC = 2
NS = 16
NW = NC * NS
BPW = B // NW
CB = 8
NCHUNK = BPW // CB


def _rowsum16(tr):
    acc = tr[pl.ds(0, 16)]
    for l in range(1, 16):
        acc = acc + tr[pl.ds(l * 16, 16)]
    return acc


def _sg_body(cw, pw, nw, w_in, w_out, pos_out, neg_out,
             ci, pi, pi2, ni, ni2, vin2, vout2, vneg2, po, no2, tr,
             sem_vin, sem_ctx):
    wid = lax.axis_index("s") * NC + lax.axis_index("c")
    wbase = wid * BPW
    lanes = lax.iota(jnp.int32, 16)
    zeros16 = lanes - lanes

    pltpu.sync_copy(cw.at[pl.ds(wbase, BPW)], ci)
    pltpu.sync_copy(pw.at[pl.ds(wbase, BPW)], pi)

    def halve_flat(v, _):
        pi2[pl.ds(v * 16, 16)] = pi[pl.ds(v * 16, 16)] // 2
        return 0

    lax.fori_loop(0, BPW // 16, halve_flat, 0)

    def stage_neg(c, pc):
        pltpu.sync_copy(nw.at[pl.ds(wbase + c * CB, CB)], ni.at[pc])

        def hrow(i, _):
            ni2[pc, i, pl.ds(KP - 16, 16)] = zeros16
            ni2[pc, i, pl.ds(0, 16)] = ni[pc, i, pl.ds(0, 16)] // 2
            ni2[pc, i, pl.ds(K - 16, 16)] = ni[pc, i, pl.ds(K - 16, 16)] // 2
            return 0

        lax.fori_loop(0, CB, hrow, 0)

    def issue(c, p):
        civ = ci[pl.ds(c * CB, 16)]
        for l in range(CB):
            pltpu.async_copy(w_in.at[civ[l] // 8], vin2.at[p, l],
                             sem_vin.at[p])
        pltpu.async_copy(w_out.at[pi2.at[pl.ds(c * CB, CB)]], vout2.at[p],
                         sem_ctx.at[p])

        def gi(i, _):
            pltpu.async_copy(w_out.at[ni2.at[p, i]], vneg2.at[p, i],
                             sem_ctx.at[p])
            return 0

        lax.fori_loop(0, CB, gi, 0)

    def drain(c, p):
        for l in range(CB):
            pltpu.make_async_copy(w_in.at[0], vin2.at[p, l],
                                  sem_vin.at[p]).wait()
        pltpu.make_async_copy(w_out.at[pi2.at[pl.ds(c * CB, CB)]],
                              vout2.at[p], sem_ctx.at[p]).wait()

        def gw(i, _):
            pltpu.make_async_copy(w_out.at[ni2.at[p, i]], vneg2.at[p, i],
                                  sem_ctx.at[p]).wait()
            return 0

        lax.fori_loop(0, CB, gw, 0)

    def compute(c, p):
        civ = ci[pl.ds(c * CB, 16)]
        piv = pi[pl.ds(c * CB, 16)]

        for l in range(CB):
            sa = civ[l] - (civ[l] // 8) * 8
            hb = (piv[l] - (piv[l] // 2) * 2) * DIM
            acc = vin2[p, l, sa, pl.ds(0, 16)] * vout2[p, l, pl.ds(hb, 16)]
            for q in range(1, 4):
                acc = acc + (vin2[p, l, sa, pl.ds(q * 16, 16)]
                             * vout2[p, l, pl.ds(hb + q * 16, 16)])
            plsc.store_scatter(tr, [lanes * 16 + l], acc)
        plsc.store_scatter(po, [c * CB + lanes], _rowsum16(tr),
                           mask=lanes < CB)

        def neg_group(g, _):
            for l in range(16):
                r = g * 16 + l
                i = r // K
                k = r - i * K
                av = ci[pl.ds(c * CB + i, 16)][0]
                sa = av - (av // 8) * 8
                bv = ni[p, i, pl.ds(k, 16)][0]
                hb = (bv - (bv // 2) * 2) * DIM
                acc = (vin2[p, i, sa, pl.ds(0, 16)]
                       * vneg2[p, i, k, pl.ds(hb, 16)])
                for q in range(1, 4):
                    acc = acc + (vin2[p, i, sa, pl.ds(q * 16, 16)]
                                 * vneg2[p, i, k, pl.ds(hb + q * 16, 16)])
                plsc.store_scatter(tr, [lanes * 16 + l], acc)
            flat = g * 16 + lanes
            rows = flat // K
            plsc.store_scatter(no2.at[p], [rows, flat - rows * K],
                               _rowsum16(tr))
            return 0

        lax.fori_loop(0, CB * K // 16, neg_group, 0)
        pltpu.sync_copy(no2.at[p], neg_out.at[pl.ds(wbase + c * CB, CB)])

    stage_neg(0, 0)
    issue(0, 0)

    def chunk_body(c, _):
        p = lax.rem(c, 2)

        @pl.when(c + 1 < NCHUNK)
        def _():
            pn = lax.rem(c + 1, 2)
            stage_neg(c + 1, pn)
            issue(c + 1, pn)

        drain(c, p)
        compute(c, p)
        return 0

    lax.fori_loop(0, NCHUNK, chunk_body, 0)
    pltpu.sync_copy(po, pos_out.at[pl.ds(wbase, BPW)])


_sg_call = functools.partial(
    pl.kernel,
    out_type=[
        jax.ShapeDtypeStruct((B,), jnp.float32),
        jax.ShapeDtypeStruct((B, K), jnp.float32),
    ],
    mesh=plsc.VectorSubcoreMesh(core_axis_name="c", subcore_axis_name="s"),
    compiler_params=pltpu.CompilerParams(
        needs_layout_passes=False,
        use_tc_tiling_on_sc=True,
        disable_bounds_checks=True,
    ),
    scratch_types=[
        pltpu.VMEM((BPW,), jnp.int32),
        pltpu.VMEM((BPW,), jnp.int32),
        pltpu.VMEM((BPW,), jnp.int32),
        pltpu.VMEM((2, CB, K), jnp.int32),
        pltpu.VMEM((2, CB, KP), jnp.int32),
        pltpu.VMEM((2, CB, 8, DIM), jnp.float32),
        pltpu.VMEM((2, CB, W2), jnp.float32),
        pltpu.VMEM((2, CB, KP, W2), jnp.float32),
        pltpu.VMEM((BPW,), jnp.float32),
        pltpu.VMEM((2, CB, K), jnp.float32),
        pltpu.VMEM((256,), jnp.float32),
        pltpu.SemaphoreType.DMA((2,)),
        pltpu.SemaphoreType.DMA((2,)),
    ],
)(_sg_body)


def kernel(center_words, pos_context_words, neg_context_words, W_in, W_out):
    cw = center_words.astype(jnp.int32)
    pw = pos_context_words.astype(jnp.int32)
    nw = neg_context_words.astype(jnp.int32)
    w_in3 = W_in.reshape(VOCAB // 8, 8, DIM)
    w_out2 = W_out.reshape(VOCAB // 2, W2)
    pos_scores, neg_scores = _sg_call(cw, pw, nw, w_in3, w_out2)
    return pos_scores, neg_scores


# speedup vs baseline: 1.0009x; 1.0009x over previous
import functools

import jax
import jax.numpy as jnp
from jax import lax
from jax.experimental import pallas as pl
from jax.experimental.pallas import tpu as pltpu
from jax.experimental.pallas import tpu_sc as plsc

VOCAB = 1000000
DIM = 64
B = 16384
K = 20
KP = 24
W2 = 2 * DIM

NC = 2
NS = 16
NW = NC * NS
BPW = B // NW
CB = 8
NCHUNK = BPW // CB


def _rowsum16(tr):
    acc = tr[pl.ds(0, 16)]
    for l in range(1, 16):
        acc = acc + tr[pl.ds(l * 16, 16)]
    return acc


def _sg_body(cw, pw, nw, w_in, w_out, pos_out, neg_out,
             ci, pi, pi2, ni, ni2, parT, vin2, vout2, vneg2, po, no2, tr,
             sem_vin, sem_ctx):
    wid = lax.axis_index("s") * NC + lax.axis_index("c")
    wbase = wid * BPW
    lanes = lax.iota(jnp.int32, 16)
    zeros16 = lanes - lanes

    pltpu.sync_copy(cw.at[pl.ds(wbase, BPW)], ci)
    pltpu.sync_copy(pw.at[pl.ds(wbase, BPW)], pi)

    def halve_flat(v, _):
        pi2[pl.ds(v * 16, 16)] = pi[pl.ds(v * 16, 16)] // 2
        return 0

    lax.fori_loop(0, BPW // 16, halve_flat, 0)

    def stage_neg(c, pc):
        pltpu.sync_copy(nw.at[pl.ds(wbase + c * CB, CB)], ni.at[pc])

        def hrow(i, _):
            ni2[pc, i, pl.ds(KP - 16, 16)] = zeros16
            lo = ni[pc, i, pl.ds(0, 16)]
            hi = ni[pc, i, pl.ds(K - 16, 16)]
            ni2[pc, i, pl.ds(0, 16)] = lo // 2
            ni2[pc, i, pl.ds(K - 16, 16)] = hi // 2
            plsc.store_scatter(parT, [pc * (K * 16) + lanes * 16 + i],
                               lo - (lo // 2) * 2)
            plsc.store_scatter(parT,
                               [pc * (K * 16) + (lanes + K - 16) * 16 + i],
                               hi - (hi // 2) * 2)
            return 0

        lax.fori_loop(0, CB, hrow, 0)

    def issue(c, p):
        civ = ci[pl.ds(c * CB, 16)]
        for l in range(CB):
            pltpu.async_copy(w_in.at[civ[l] // 8], vin2.at[p, l],
                             sem_vin.at[p])
        pltpu.async_copy(w_out.at[pi2.at[pl.ds(c * CB, CB)]], vout2.at[p],
                         sem_ctx.at[p])

        def gi(i, _):
            pltpu.async_copy(w_out.at[ni2.at[p, i]], vneg2.at[p, i],
                             sem_ctx.at[p])
            return 0

        lax.fori_loop(0, CB, gi, 0)

    def drain(c, p):
        for l in range(CB):
            pltpu.make_async_copy(w_in.at[0], vin2.at[p, l],
                                  sem_vin.at[p]).wait()
        pltpu.make_async_copy(w_out.at[pi2.at[pl.ds(c * CB, CB)]],
                              vout2.at[p], sem_ctx.at[p]).wait()

        def gw(i, _):
            pltpu.make_async_copy(w_out.at[ni2.at[p, i]], vneg2.at[p, i],
                                  sem_ctx.at[p]).wait()
            return 0

        lax.fori_loop(0, CB, gw, 0)

    def compute(c, p):
        civ = ci[pl.ds(c * CB, 16)]
        piv = pi[pl.ds(c * CB, 16)]
        sv = civ - (civ // 8) * 8
        ppar = piv - (piv // 2) * 2

        def dot(a_l, sa, b_slices, par_l):
            acc_e = vin2[p, a_l, sa, pl.ds(0, 16)] * b_slices[0]
            acc_o = vin2[p, a_l, sa, pl.ds(0, 16)] * b_slices[4]
            for q in range(1, 4):
                a_q = vin2[p, a_l, sa, pl.ds(q * 16, 16)]
                acc_e = acc_e + a_q * b_slices[q]
                acc_o = acc_o + a_q * b_slices[4 + q]
            return jnp.where(par_l == 0, acc_e, acc_o)

        for l in range(CB):
            bs = [vout2[p, l, pl.ds(q * 16, 16)] for q in range(8)]
            plsc.store_scatter(tr, [lanes * 16 + l],
                               dot(l, sv[l], bs, ppar[l]))
        plsc.store_scatter(po, [c * CB + lanes], _rowsum16(tr),
                           mask=lanes < CB)

        def neg_group(k, _):
            parv = parT[pl.ds(p * (K * 16) + k * 16, 16)]
            for l in range(CB):
                bs = [vneg2[p, l, k, pl.ds(q * 16, 16)] for q in range(8)]
                plsc.store_scatter(tr, [lanes * 16 + l],
                                   dot(l, sv[l], bs, parv[l]))
            plsc.store_scatter(no2.at[p], [lanes, zeros16 + k],
                               _rowsum16(tr), mask=lanes < CB)
            return 0

        lax.fori_loop(0, K, neg_group, 0)
        pltpu.sync_copy(no2.at[p], neg_out.at[pl.ds(wbase + c * CB, CB)])

    stage_neg(0, 0)
    issue(0, 0)

    def chunk_body(c, _):
        p = lax.rem(c, 2)

        @pl.when(c + 1 < NCHUNK)
        def _():
            pn = lax.rem(c + 1, 2)
            stage_neg(c + 1, pn)
            issue(c + 1, pn)

        drain(c, p)
        compute(c, p)
        return 0

    lax.fori_loop(0, NCHUNK, chunk_body, 0)
    pltpu.sync_copy(po, pos_out.at[pl.ds(wbase, BPW)])


_sg_call = functools.partial(
    pl.kernel,
    out_type=[
        jax.ShapeDtypeStruct((B,), jnp.float32),
        jax.ShapeDtypeStruct((B, K), jnp.float32),
    ],
    mesh=plsc.VectorSubcoreMesh(core_axis_name="c", subcore_axis_name="s"),
    compiler_params=pltpu.CompilerParams(
        needs_layout_passes=False,
        use_tc_tiling_on_sc=True,
        disable_bounds_checks=True,
    ),
    scratch_types=[
        pltpu.VMEM((BPW,), jnp.int32),
        pltpu.VMEM((BPW,), jnp.int32),
        pltpu.VMEM((BPW,), jnp.int32),
        pltpu.VMEM((2, CB, K), jnp.int32),
        pltpu.VMEM((2, CB, KP), jnp.int32),
        pltpu.VMEM((2 * K * 16,), jnp.int32),
        pltpu.VMEM((2, CB, 8, DIM), jnp.float32),
        pltpu.VMEM((2, CB, W2), jnp.float32),
        pltpu.VMEM((2, CB, KP, W2), jnp.float32),
        pltpu.VMEM((BPW,), jnp.float32),
        pltpu.VMEM((2, CB, K), jnp.float32),
        pltpu.VMEM((256,), jnp.float32),
        pltpu.SemaphoreType.DMA((2,)),
        pltpu.SemaphoreType.DMA((2,)),
    ],
)(_sg_body)


def kernel(center_words, pos_context_words, neg_context_words, W_in, W_out):
    cw = center_words.astype(jnp.int32)
    pw = pos_context_words.astype(jnp.int32)
    nw = neg_context_words.astype(jnp.int32)
    w_in3 = W_in.reshape(VOCAB // 8, 8, DIM)
    w_out2 = W_out.reshape(VOCAB // 2, W2)
    pos_scores, neg_scores = _sg_call(cw, pw, nw, w_in3, w_out2)
    return pos_scores, neg_scores


# speedup vs baseline: 2.6797x; 2.6773x over previous
import functools

import jax
import jax.numpy as jnp
from jax import lax
from jax.experimental import pallas as pl
from jax.experimental.pallas import tpu as pltpu
from jax.experimental.pallas import tpu_sc as plsc

VOCAB = 1000000
DIM = 64
B = 16384
K = 20

NC = 2
NS = 16
NW = NC * NS
BPW = B // NW
CB = 32
NCHUNK = BPW // CB


def _partial64(a_ref, arow, b_ref, brow):
    acc = a_ref[arow, pl.ds(0, 16)] * b_ref[brow, pl.ds(0, 16)]
    for j in range(1, 4):
        acc = acc + a_ref[arow, pl.ds(j * 16, 16)] * b_ref[brow, pl.ds(j * 16, 16)]
    return acc


def _rowsum16(tr):
    acc = tr[pl.ds(0, 16)]
    for l in range(1, 16):
        acc = acc + tr[pl.ds(l * 16, 16)]
    return acc


def _sg_body(cw, pw, nw, w_in, w_out, pos_out, neg_out,
             ci, pi, ni, vin2, vout2, vneg2, po, no, tr, sems):
    wid = lax.axis_index("s") * NC + lax.axis_index("c")
    wbase = wid * BPW

    pltpu.sync_copy(cw.at[pl.ds(wbase, BPW)], ci)
    pltpu.sync_copy(pw.at[pl.ds(wbase, BPW)], pi)
    pltpu.sync_copy(nw.at[pl.ds(wbase, BPW)], ni)

    def issue(c, p):
        sem = sems.at[p]
        pltpu.async_copy(w_in.at[ci.at[pl.ds(c * CB, CB)]], vin2.at[p], sem)
        pltpu.async_copy(w_out.at[pi.at[pl.ds(c * CB, CB)]], vout2.at[p], sem)

        def gi(i, _):
            pltpu.async_copy(w_out.at[ni.at[c * CB + i]], vneg2.at[p, i], sem)
            return 0

        lax.fori_loop(0, CB, gi, 0)

    def drain(c, p):
        sem = sems.at[p]
        pltpu.make_async_copy(w_in.at[ci.at[pl.ds(c * CB, CB)]], vin2.at[p], sem).wait()
        pltpu.make_async_copy(w_out.at[pi.at[pl.ds(c * CB, CB)]], vout2.at[p], sem).wait()

        def gw(i, _):
            pltpu.make_async_copy(w_out.at[ni.at[c * CB + i]], vneg2.at[p, i], sem).wait()
            return 0

        lax.fori_loop(0, CB, gw, 0)

    def compute(c, p):
        lanes = lax.iota(jnp.int32, 16)

        def pos_group(g, _):
            for l in range(16):
                r = g * 16 + l
                plsc.store_scatter(tr, [lanes * 16 + l],
                                   _partial64(vin2.at[p], r, vout2.at[p], r))
            po[pl.ds(c * CB + g * 16, 16)] = _rowsum16(tr)
            return 0

        lax.fori_loop(0, CB // 16, pos_group, 0)

        def neg_group(g, _):
            for l in range(16):
                r = g * 16 + l
                i = r // K
                k = r - i * K
                a = vin2.at[p]
                b = vneg2.at[p]
                acc = a[i, pl.ds(0, 16)] * b[i, k, pl.ds(0, 16)]
                for j in range(1, 4):
                    acc = acc + a[i, pl.ds(j * 16, 16)] * b[i, k, pl.ds(j * 16, 16)]
                plsc.store_scatter(tr, [lanes * 16 + l], acc)
            flat = c * CB * K + g * 16 + lanes
            rows = flat // K
            plsc.store_scatter(no, [rows, flat - rows * K], _rowsum16(tr))
            return 0

        lax.fori_loop(0, CB * K // 16, neg_group, 0)

    issue(0, 0)

    def chunk_body(c, _):
        p = lax.rem(c, 2)

        @pl.when(c + 1 < NCHUNK)
        def _():
            issue(c + 1, lax.rem(c + 1, 2))

        drain(c, p)
        compute(c, p)
        return 0

    lax.fori_loop(0, NCHUNK, chunk_body, 0)
    pltpu.sync_copy(po, pos_out.at[pl.ds(wbase, BPW)])
    pltpu.sync_copy(no, neg_out.at[pl.ds(wbase, BPW)])


_sg_call = functools.partial(
    pl.kernel,
    out_type=[
        jax.ShapeDtypeStruct((B,), jnp.float32),
        jax.ShapeDtypeStruct((B, K), jnp.float32),
    ],
    mesh=plsc.VectorSubcoreMesh(core_axis_name="c", subcore_axis_name="s"),
    compiler_params=pltpu.CompilerParams(
        needs_layout_passes=False, use_tc_tiling_on_sc=False
    ),
    scratch_types=[
        pltpu.VMEM((BPW,), jnp.int32),
        pltpu.VMEM((BPW,), jnp.int32),
        pltpu.VMEM((BPW, K), jnp.int32),
        pltpu.VMEM((2, CB, DIM), jnp.float32),
        pltpu.VMEM((2, CB, DIM), jnp.float32),
        pltpu.VMEM((2, CB, K, DIM), jnp.float32),
        pltpu.VMEM((BPW,), jnp.float32),
        pltpu.VMEM((BPW, K), jnp.float32),
        pltpu.VMEM((256,), jnp.float32),
        pltpu.SemaphoreType.DMA((2,)),
    ],
)(_sg_body)


def kernel(center_words, pos_context_words, neg_context_words, W_in, W_out):
    cw = center_words.astype(jnp.int32)
    pw = pos_context_words.astype(jnp.int32)
    nw = neg_context_words.astype(jnp.int32)
    pos_scores, neg_scores = _sg_call(cw, pw, nw, W_in, W_out)
    return pos_scores, neg_scores
